# baseline (speedup 1.0000x reference)
import functools

import jax
import jax.numpy as jnp
from jax import lax
from jax.experimental import pallas as pl
from jax.experimental.pallas import tpu as pltpu
from jax.experimental.pallas import tpu_sc as plsc

_D = 64
_B = 16384
_NC = 2
_NS = 16
_NW = _NC * _NS
_CHUNK = 128
_RW = _B // _NW
_K = _RW // _CHUNK

_mesh = plsc.VectorSubcoreMesh(core_axis_name="c", subcore_axis_name="s")


@functools.partial(
    pl.kernel,
    out_type=jax.ShapeDtypeStruct((_B, _D), jnp.float32),
    mesh=_mesh,
    compiler_params=pltpu.CompilerParams(use_tc_tiling_on_sc=False),
    scratch_types=[
        pltpu.VMEM((_K, _CHUNK), jnp.int32),
        pltpu.VMEM((_RW, _D), jnp.float32),
        pltpu.SemaphoreType.DMA,
    ],
)
def _one_gather(table, idx2, out, idx_v, rows, sem):
    wid = lax.axis_index("s") * _NC + lax.axis_index("c")
    base = wid * _K
    pltpu.sync_copy(idx2.at[pl.ds(base, _K)], idx_v)
    copies = []
    for j in range(_K):
        copies.append(
            pltpu.async_copy(
                table.at[idx_v.at[j]], rows.at[pl.ds(j * _CHUNK, _CHUNK)], sem
            )
        )
    for c in copies:
        c.wait()
    pltpu.sync_copy(rows, out.at[pl.ds(wid * _RW, _RW)])


@jax.jit
def kernel(ori, dest, table):
    ori2 = ori.reshape(_B // _CHUNK, _CHUNK)
    dest2 = dest.reshape(_B // _CHUNK, _CHUNK)
    return _one_gather(table, ori2), _one_gather(table, dest2)


# speedup vs baseline: 1.7128x; 1.7128x over previous
import functools

import jax
import jax.numpy as jnp
from jax import lax
from jax.experimental import pallas as pl
from jax.experimental.pallas import tpu as pltpu
from jax.experimental.pallas import tpu_sc as plsc

_D = 64
_B = 16384
_NC = 2
_NS = 16
_NW = _NC * _NS
_ROWS_PER_W = _B // _NW
_CHUNK = 256
_K = _ROWS_PER_W // _CHUNK
_NSEM = 4

_mesh = plsc.VectorSubcoreMesh(core_axis_name="c", subcore_axis_name="s")


@functools.partial(
    pl.kernel,
    out_type=(
        jax.ShapeDtypeStruct((_B, _D), jnp.float32),
        jax.ShapeDtypeStruct((_B, _D), jnp.float32),
    ),
    mesh=_mesh,
    scratch_types=[
        pltpu.VMEM((_ROWS_PER_W,), jnp.int32),
        pltpu.VMEM((_ROWS_PER_W,), jnp.int32),
        pltpu.VMEM((_CHUNK, _D), jnp.float32),
        pltpu.VMEM((_CHUNK, _D), jnp.float32),
        [pltpu.SemaphoreType.DMA] * _NSEM,
        [pltpu.SemaphoreType.DMA] * _NSEM,
    ],
)
def _od_gather(table, ori, dest, o_out, d_out, oidx_v, didx_v, obuf, dbuf, sems_o, sems_d):
    wid = lax.axis_index("s") * _NC + lax.axis_index("c")
    row0 = wid * _ROWS_PER_W
    pltpu.sync_copy(ori.at[pl.ds(row0, _ROWS_PER_W)], oidx_v)
    pltpu.sync_copy(dest.at[pl.ds(row0, _ROWS_PER_W)], didx_v)

    def chunk_body(c, _):
        def group_body(g, _):
            ovec = oidx_v[pl.ds(c * _CHUNK + g * 16, 16)]
            dvec = didx_v[pl.ds(c * _CHUNK + g * 16, 16)]
            for l in range(16):
                pltpu.async_copy(
                    table.at[pl.ds(ovec[l], 1)],
                    obuf.at[pl.ds(g * 16 + l, 1)],
                    sems_o[l % _NSEM],
                )
                pltpu.async_copy(
                    table.at[pl.ds(dvec[l], 1)],
                    dbuf.at[pl.ds(g * 16 + l, 1)],
                    sems_d[l % _NSEM],
                )
            return ()

        lax.fori_loop(0, _CHUNK // 16, group_body, ())
        n_per_sem = _CHUNK // _NSEM
        for s in range(_NSEM):
            pltpu.make_async_copy(
                table.at[pl.ds(0, n_per_sem)], obuf.at[pl.ds(0, n_per_sem)], sems_o[s]
            ).wait()
            pltpu.make_async_copy(
                table.at[pl.ds(0, n_per_sem)], dbuf.at[pl.ds(0, n_per_sem)], sems_d[s]
            ).wait()
        pltpu.sync_copy(obuf, o_out.at[pl.ds(row0 + c * _CHUNK, _CHUNK)])
        pltpu.sync_copy(dbuf, d_out.at[pl.ds(row0 + c * _CHUNK, _CHUNK)])
        return ()

    lax.fori_loop(0, _K, chunk_body, ())


@jax.jit
def kernel(ori, dest, table):
    return _od_gather(table, ori, dest)
